# Initial kernel scaffold; baseline (speedup 1.0000x reference)
import jax
import jax.numpy as jnp
from jax.experimental import pallas as pl


def kernel(proto_types, map_class, map_box, map_coef, anchor_center, anchor_box, gt_boxes, gt_masks, anchor_class, gt_idx):
    raise NotImplementedError("write your pallas kernel here")


# speedup vs baseline: 35.6738x; 35.6738x over previous
import jax
import jax.numpy as jnp
from jax import lax
from jax.experimental import pallas as pl
from jax.experimental.pallas import tpu as pltpu

_P = 4
_ALPHA = 1.0
_ZCLIP = 13.8155096

_LANE = 128
_CHUNK = 128
_SLAB = 32


def _tc_kernel(proto_ref, cls_ref, box_ref, coef_ref, ctr_ref, abox_ref,
               gtb_ref, gtm_ref, ac_ref, g4_ref, out_ref, klist_ref):
    n_img = proto_ref.shape[0]
    n_rows = ac_ref.shape[1]
    n_anch = n_rows * _LANE
    a_num = g4_ref.shape[1]
    rows_per_a = g4_ref.shape[2]
    hm, wm = gtm_ref.shape[2], gtm_ref.shape[3]
    inv_px = 1.0 / float(hm * wm)
    inv_ln10 = 0.43429448190325176

    r0 = lax.broadcasted_iota(jnp.int32, (_LANE, _LANE), 0)
    c0 = lax.broadcasted_iota(jnp.int32, (_LANE, _LANE), 1)
    tri_incl = (r0 <= c0).astype(jnp.float32)
    rr0 = lax.broadcasted_iota(jnp.int32, (n_rows, n_rows), 0)
    cc0 = lax.broadcasted_iota(jnp.int32, (n_rows, n_rows), 1)
    tri_strict = (cc0 < rr0).astype(jnp.float32)

    lane_iota = lax.broadcasted_iota(jnp.int32, (1, _LANE), 1)

    total = jnp.float32(0.0)
    for i in range(n_img):
        ac = ac_ref[i]
        posf = (ac != 0).astype(jnp.float32)
        negf = 1.0 - posf
        npos = jnp.sum(posf)
        has_pos = npos > 0.0
        npos_f = jnp.where(has_pos, npos, 1.0)
        total_neg = float(n_anch) - npos
        nneg = jnp.minimum(3.0 * npos, total_neg)
        nneg_f = jnp.where(nneg > 0.0, nneg, 1.0)

        rc_pos = lax.dot(posf, tri_incl,
                         precision=lax.Precision.HIGHEST,
                         preferred_element_type=jnp.float32)
        rc_neg = lax.dot(negf, tri_incl,
                         precision=lax.Precision.HIGHEST,
                         preferred_element_type=jnp.float32)
        offs_pos = lax.dot(tri_strict, rc_pos[:, _LANE - 1:_LANE],
                           precision=lax.Precision.HIGHEST,
                           preferred_element_type=jnp.float32)
        offs_neg = lax.dot(tri_strict, rc_neg[:, _LANE - 1:_LANE],
                           precision=lax.Precision.HIGHEST,
                           preferred_element_type=jnp.float32)
        poscum = rc_pos + offs_pos
        negcum = rc_neg + offs_neg

        p = cls_ref[i]
        p_c = jnp.clip(p, 1e-6, 1.0 - 1e-6)
        neg_sel = negf * (negcum <= nneg).astype(jnp.float32)
        l_cls_pos = jnp.sum(posf * (-jnp.log(p_c))) / npos_f
        l_cls_neg = jnp.sum(neg_sel * (-jnp.log(1.0 - p_c))) / nneg_f

        ach = ctr_ref[0]
        acw = ctr_ref[1]
        l_loc = jnp.float32(0.0)
        for a in range(a_num):
            g_a = g4_ref[i, a]
            pos_a = posf[a * rows_per_a:(a + 1) * rows_per_a, :]
            a_h = abox_ref[a, 0]
            a_w = abox_ref[a, 1]
            gts = []
            for x in range(4):
                acc = jnp.zeros((rows_per_a, _LANE), jnp.float32)
                for j in range(gtb_ref.shape[1]):
                    acc = jnp.where(g_a == j, gtb_ref[i, j, x], acc)
                gts.append(acc)
            t0 = (gts[0] - ach) / a_h
            t1 = (gts[1] - acw) / a_w
            t2 = jnp.log(gts[2] / a_h) * inv_ln10
            t3 = jnp.log(gts[3] / a_w) * inv_ln10
            for x, t in enumerate((t0, t1, t2, t3)):
                pr = box_ref[i, a * 4 + x]
                d = jnp.abs(pr - t)
                sl = jnp.where(d < 1.0, 0.5 * d * d, d - 0.5)
                l_loc = l_loc + jnp.sum(pos_a * sl)

        protos = [proto_ref[i, pp] for pp in range(_P)]
        pc3 = poscum.reshape(1, n_rows, _LANE)
        ni = npos.astype(jnp.int32)
        nchunks = (ni + (_CHUNK - 1)) // _CHUNK

        def chunk_body(c, l_msk):
            base = c * _CHUNK
            basef = base.astype(jnp.float32)
            for s in range(_CHUNK // _SLAB):
                thr = (basef + float(s * _SLAB) + 1.0 +
                       lax.broadcasted_iota(
                           jnp.int32, (_SLAB, 1, 1), 0).astype(jnp.float32))
                cnt = jnp.sum((pc3 < thr).astype(jnp.float32), axis=(1, 2),
                              keepdims=True)
                klist_ref[s * _SLAB:(s + 1) * _SLAB, :] = cnt.reshape(_SLAB, 1)
            jmax = jnp.minimum(_CHUNK, ni - base)

            def pos_body(j, acc):
                k = klist_ref[pl.ds(j, 1), :][0, 0].astype(jnp.int32)
                a = k // (rows_per_a * _LANE)
                rem = k - a * (rows_per_a * _LANE)
                rw = rem // _LANE
                cl = rem - rw * _LANE
                onehot = (lane_iota == cl)
                grow = g4_ref[i, a, pl.ds(rw, 1), :]
                g = jnp.sum(jnp.where(onehot, grow, 0))
                z = jnp.zeros((hm, wm), jnp.float32)
                for pp in range(_P):
                    crow = coef_ref[i, a * _P + pp, pl.ds(rw, 1), :]
                    cval = jnp.sum(jnp.where(onehot, crow, 0.0))
                    z = z + cval * protos[pp]
                zc = jnp.clip(z, -_ZCLIP, _ZCLIP)
                sp = jnp.maximum(zc, 0.0) + jnp.log1p(jnp.exp(-jnp.abs(zc)))
                y = gtm_ref[i, g]
                lm = (jnp.sum(sp) - jnp.sum(y * zc)) * inv_px
                return acc + lm

            return lax.fori_loop(0, jmax, pos_body, l_msk)

        l_msk = lax.fori_loop(0, nchunks, chunk_body, jnp.float32(0.0))

        total = total + jnp.where(
            has_pos,
            (l_cls_pos + l_cls_neg) / npos_f
            + _ALPHA * l_loc / npos_f
            + l_msk / npos_f,
            0.0)

    out_ref[:, :] = jnp.broadcast_to(total, (1, 1))


def kernel(proto_types, map_class, map_box, map_coef, anchor_center,
           anchor_box, gt_boxes, gt_masks, anchor_class, gt_idx):
    n, a_num, h, w = anchor_class.shape
    n_rows = a_num * h * w // _LANE
    rows_per_a = h * w // _LANE
    cls2 = map_class.reshape(n, n_rows, _LANE)
    box4 = map_box.reshape(n, a_num * 4, rows_per_a, _LANE)
    coef4 = map_coef.reshape(n, a_num * _P, rows_per_a, _LANE)
    ctr = anchor_center.reshape(2, rows_per_a, _LANE)
    ac2 = anchor_class.reshape(n, n_rows, _LANE)
    g4 = gt_idx.reshape(n, a_num, rows_per_a, _LANE)
    out = pl.pallas_call(
        _tc_kernel,
        out_shape=jax.ShapeDtypeStruct((1, 1), jnp.float32),
        scratch_shapes=[pltpu.VMEM((_CHUNK, 1), jnp.float32)],
    )(proto_types, cls2, box4, coef4, ctr, anchor_box, gt_boxes,
      gt_masks, ac2, g4)
    return out.reshape(())
